# Initial kernel scaffold; baseline (speedup 1.0000x reference)
import jax
import jax.numpy as jnp
from jax.experimental import pallas as pl


def kernel(x, edge_index, W1, b1, W2, b2, W3, b3, W4, b4):
    raise NotImplementedError("write your pallas kernel here")


# speedup vs baseline: 10.4237x; 10.4237x over previous
import functools

import jax
import jax.numpy as jnp
from jax import lax
from jax.experimental import pallas as pl
from jax.experimental.pallas import tpu as pltpu
from jax.experimental.pallas import tpu_sc as plsc

NC = 2
NS = 16
LANES = 16
EB = 128
ROWBLK = 1000


def _split_blocks(nblk, nworkers, w):
    base, extra = nblk // nworkers, nblk % nworkers
    cnt = base + jnp.where(w < extra, 1, 0)
    lo = w * base + jnp.minimum(w, extra)
    return lo, cnt


def _make_degree(N, E):
    assert E % EB == 0
    nblk = E // EB
    mesh = plsc.VectorSubcoreMesh(core_axis_name="c", subcore_axis_name="s")

    @functools.partial(
        pl.kernel, mesh=mesh,
        out_type=jax.ShapeDtypeStruct((NC * NS, N), jnp.float32),
        compiler_params=pltpu.CompilerParams(needs_layout_passes=False),
        scratch_types=[
            pltpu.VMEM((N,), jnp.float32),
            pltpu.VMEM((EB,), jnp.int32),
        ],
    )
    def deg_kernel(dst_hbm, out_hbm, dloc, blk):
        c = lax.axis_index("c")
        s = lax.axis_index("s")
        wid = c * NS + s

        def zero_step(i, _):
            dloc[pl.ds(i * LANES, LANES)] = jnp.zeros((LANES,), jnp.float32)
            return 0
        lax.fori_loop(0, N // LANES, zero_step, 0)

        lo, cnt = _split_blocks(nblk, NC * NS, wid)
        ones = jnp.ones((LANES,), jnp.float32)

        def step(i, _):
            pltpu.sync_copy(dst_hbm.at[pl.ds((lo + i) * EB, EB)], blk)
            for j in range(EB // LANES):
                idx = blk[pl.ds(j * LANES, LANES)]
                plsc.addupdate_scatter(dloc, [idx], ones)
            return 0
        lax.fori_loop(0, cnt, step, 0)

        pltpu.sync_copy(dloc, out_hbm.at[wid])

    return deg_kernel


def _make_agg(N, E, C, Fc):
    assert E % EB == 0 and N % NS == 0
    nblk = E // EB
    rpt = N // NS
    cpc = max(C // NC, 1)
    out_shape = (jax.ShapeDtypeStruct((C * N, Fc), jnp.float32) if C >= 2
                 else jax.ShapeDtypeStruct((NC, N, Fc), jnp.float32))
    mesh = plsc.VectorSubcoreMesh(core_axis_name="c", subcore_axis_name="s")

    @functools.partial(
        pl.kernel, mesh=mesh,
        out_type=out_shape,
        compiler_params=pltpu.CompilerParams(use_tc_tiling_on_sc=False),
        scratch_types=[
            pltpu.VMEM_SHARED((N, Fc), jnp.float32),
            pltpu.VMEM((EB,), jnp.int32),
            pltpu.VMEM((EB,), jnp.int32),
            pltpu.VMEM((EB,), jnp.int32),
            pltpu.VMEM((EB, Fc), jnp.float32),
            pltpu.SemaphoreType.DMA,
        ],
    )
    def agg_kernel(z_hbm, src_hbm, dst_hbm, y_hbm, acc, bsrc, bdst, bidx,
                   rows, sem):
        cid = lax.axis_index("c")
        sid = lax.axis_index("s")

        for ci in range(cpc):
            if C >= 2:
                chunk = cid * cpc + ci
                row0 = chunk * N
                pltpu.sync_copy(z_hbm.at[pl.ds(row0 + sid * rpt, rpt)],
                                acc.at[pl.ds(sid * rpt, rpt)])
                blo, bcnt = _split_blocks(nblk, NS, sid)
            else:
                row0 = jnp.int32(0)
                def zstep(i, _):
                    for j in range(Fc // LANES):
                        rows[i, pl.ds(j * LANES, LANES)] = (
                            jnp.zeros((LANES,), jnp.float32))
                    return 0
                lax.fori_loop(0, EB, zstep, 0)
                nfull, tail = rpt // EB, rpt % EB
                for k in range(nfull):
                    pltpu.sync_copy(
                        rows, acc.at[pl.ds(sid * rpt + k * EB, EB)])
                if tail:
                    pltpu.sync_copy(
                        rows.at[pl.ds(0, tail)],
                        acc.at[pl.ds(sid * rpt + nfull * EB, tail)])
                blo, bcnt = _split_blocks(nblk, NC * NS, cid * NS + sid)

            plsc.subcore_barrier()

            def step(i, _):
                e0 = (blo + i) * EB
                pltpu.sync_copy(src_hbm.at[pl.ds(e0, EB)], bsrc)
                pltpu.sync_copy(dst_hbm.at[pl.ds(e0, EB)], bdst)
                if C >= 2:
                    for j in range(EB // LANES):
                        bidx[pl.ds(j * LANES, LANES)] = (
                            bsrc[pl.ds(j * LANES, LANES)] + row0)
                    gidx = bidx
                else:
                    gidx = bsrc
                pltpu.async_copy(z_hbm.at[gidx], rows, sem).wait()
                pltpu.sync_copy(rows, acc.at[bdst], add=True)
                return 0
            lax.fori_loop(0, bcnt, step, 0)

            plsc.subcore_barrier()

            if C >= 2:
                pltpu.sync_copy(acc.at[pl.ds(sid * rpt, rpt)],
                                y_hbm.at[pl.ds(row0 + sid * rpt, rpt)])
            else:
                pltpu.sync_copy(acc.at[pl.ds(sid * rpt, rpt)],
                                y_hbm.at[cid, pl.ds(sid * rpt, rpt)])

    return agg_kernel


def _tc_call(body, grid, in_specs, out_specs, out_shape):
    return pl.pallas_call(
        body, grid=grid, in_specs=in_specs, out_specs=out_specs,
        out_shape=out_shape)


def _dinv_from(dp):
    return lax.rsqrt(jnp.sum(dp, axis=1, keepdims=True) + 1.0)


def _tc1(dpT, x, W1, N, R):
    H = W1.shape[1]
    Fc = H // 2

    def body(dp_ref, x_ref, w_ref, g1_ref):
        dv = _dinv_from(dp_ref[...])
        g = jnp.dot(x_ref[...] * dv, w_ref[...],
                    preferred_element_type=jnp.float32)
        g1_ref[0] = g[:, :Fc]
        g1_ref[1] = g[:, Fc:]

    return _tc_call(
        body, (N // R,),
        [pl.BlockSpec((R, NC * NS), lambda i: (i, 0)),
         pl.BlockSpec((R, x.shape[1]), lambda i: (i, 0)),
         pl.BlockSpec(W1.shape, lambda i: (0, 0))],
        pl.BlockSpec((2, R, Fc), lambda i: (0, i, 0)),
        jax.ShapeDtypeStruct((2, N, Fc), jnp.float32),
    )(dpT, x, W1)


def _tc2(a1, dpT, b1, N, R):
    Fc = a1.shape[2]
    b1c = b1.reshape(2, 1, Fc)

    def body(a_ref, dp_ref, b_ref, o_ref):
        dv = _dinv_from(dp_ref[...])
        for c in range(2):
            o_ref[c] = dv * jnp.maximum(dv * a_ref[c] + b_ref[c], 0.0)

    return _tc_call(
        body, (N // R,),
        [pl.BlockSpec((2, R, Fc), lambda i: (0, i, 0)),
         pl.BlockSpec((R, NC * NS), lambda i: (i, 0)),
         pl.BlockSpec((2, 1, Fc), lambda i: (0, 0, 0))],
        pl.BlockSpec((2, R, Fc), lambda i: (0, i, 0)),
        jax.ShapeDtypeStruct((2, N, Fc), jnp.float32),
    )(a1, dpT, b1c)


def _tc3(a2, dpT, W2, b2, W3, N, R):
    Fc_in = a2.shape[2]
    H3 = W3.shape[1]
    Fc = H3 // 4

    def body(a_ref, dp_ref, w2_ref, b2_ref, w3_ref, o_ref):
        dv = _dinv_from(dp_ref[...])
        h = jnp.concatenate([a_ref[0], a_ref[1]], axis=1) * dv
        t = jnp.maximum(jnp.dot(h, w2_ref[...],
                                preferred_element_type=jnp.float32)
                        + b2_ref[...], 0.0)
        g = jnp.dot(t * dv, w3_ref[...], preferred_element_type=jnp.float32)
        for c in range(4):
            o_ref[c] = g[:, c * Fc:(c + 1) * Fc]

    return _tc_call(
        body, (N // R,),
        [pl.BlockSpec((2, R, Fc_in), lambda i: (0, i, 0)),
         pl.BlockSpec((R, NC * NS), lambda i: (i, 0)),
         pl.BlockSpec(W2.shape, lambda i: (0, 0)),
         pl.BlockSpec((1, W2.shape[1]), lambda i: (0, 0)),
         pl.BlockSpec(W3.shape, lambda i: (0, 0))],
        pl.BlockSpec((4, R, Fc), lambda i: (0, i, 0)),
        jax.ShapeDtypeStruct((4, N, Fc), jnp.float32),
    )(a2, dpT, W2, b2.reshape(1, -1), W3)


def _tc4(a3, dpT, b3, W4, N, R):
    Fc = a3.shape[2]
    OUT = W4.shape[1]
    b3c = b3.reshape(4, 1, Fc)

    def body(a_ref, dp_ref, b_ref, w4_ref, o_ref):
        dv = _dinv_from(dp_ref[...])
        h = jnp.concatenate(
            [jnp.maximum(dv * a_ref[c] + b_ref[c], 0.0) for c in range(4)],
            axis=1)
        o_ref[...] = jnp.dot(h * dv, w4_ref[...],
                             preferred_element_type=jnp.float32)

    return _tc_call(
        body, (N // R,),
        [pl.BlockSpec((4, R, Fc), lambda i: (0, i, 0)),
         pl.BlockSpec((R, NC * NS), lambda i: (i, 0)),
         pl.BlockSpec((4, 1, Fc), lambda i: (0, 0, 0)),
         pl.BlockSpec(W4.shape, lambda i: (0, 0))],
        pl.BlockSpec((R, OUT), lambda i: (i, 0)),
        jax.ShapeDtypeStruct((N, OUT), jnp.float32),
    )(a3, dpT, b3c, W4)


def _tc5(parts, g4, dpT, b4, N, R):
    OUT = g4.shape[1]

    def body(p_ref, g_ref, dp_ref, b_ref, o_ref):
        dv = _dinv_from(dp_ref[...])
        o = dv * (p_ref[0] + p_ref[1] + g_ref[...]) + b_ref[...]
        m = jnp.max(o, axis=1, keepdims=True)
        e = o - m
        o_ref[...] = e - jnp.log(jnp.sum(jnp.exp(e), axis=1, keepdims=True))

    return _tc_call(
        body, (N // R,),
        [pl.BlockSpec((2, R, OUT), lambda i: (0, i, 0)),
         pl.BlockSpec((R, OUT), lambda i: (i, 0)),
         pl.BlockSpec((R, NC * NS), lambda i: (i, 0)),
         pl.BlockSpec((1, OUT), lambda i: (0, 0))],
        pl.BlockSpec((R, OUT), lambda i: (i, 0)),
        jax.ShapeDtypeStruct((N, OUT), jnp.float32),
    )(parts, g4, dpT, b4.reshape(1, -1))


def kernel(x, edge_index, W1, b1, W2, b2, W3, b3, W4, b4):
    N, DIN = x.shape
    E = edge_index.shape[1]
    R = ROWBLK
    src = edge_index[0]
    dst = edge_index[1]

    deg_parts = _make_degree(N, E)(dst)
    dpT = jnp.transpose(deg_parts)
    g1 = _tc1(dpT, x, W1, N, R)

    agg128 = _make_agg(N, E, 2, W1.shape[1] // 2)
    a1 = agg128(g1.reshape(2 * N, -1), src, dst)
    a1 = a1.reshape(2, N, -1)

    z2 = _tc2(a1, dpT, b1, N, R)
    a2 = agg128(z2.reshape(2 * N, -1), src, dst).reshape(2, N, -1)

    g3 = _tc3(a2, dpT, W2, b2, W3, N, R)
    agg512 = _make_agg(N, E, 4, W3.shape[1] // 4)
    a3 = agg512(g3.reshape(4 * N, -1), src, dst).reshape(4, N, -1)

    g4 = _tc4(a3, dpT, b3, W4, N, R)
    agg16 = _make_agg(N, E, 1, W4.shape[1])
    parts = agg16(g4, src, dst)

    return _tc5(parts, g4, dpT, b4, N, R)


# speedup vs baseline: 20.8938x; 2.0044x over previous
import functools

import jax
import jax.numpy as jnp
from jax import lax
from jax.experimental import pallas as pl
from jax.experimental.pallas import tpu as pltpu
from jax.experimental.pallas import tpu_sc as plsc

NC = 2
NS = 16
LANES = 16
EB = 128
ROWBLK = 1000


def _split_blocks(nblk, nworkers, w):
    base, extra = nblk // nworkers, nblk % nworkers
    cnt = base + jnp.where(w < extra, 1, 0)
    lo = w * base + jnp.minimum(w, extra)
    return lo, cnt


def _make_degree(N, E):
    assert E % EB == 0
    nblk = E // EB
    mesh = plsc.VectorSubcoreMesh(core_axis_name="c", subcore_axis_name="s")

    @functools.partial(
        pl.kernel, mesh=mesh,
        out_type=jax.ShapeDtypeStruct((NC * NS, N), jnp.float32),
        compiler_params=pltpu.CompilerParams(needs_layout_passes=False),
        scratch_types=[
            pltpu.VMEM((N,), jnp.float32),
            pltpu.VMEM((EB,), jnp.int32),
        ],
    )
    def deg_kernel(dst_hbm, out_hbm, dloc, blk):
        c = lax.axis_index("c")
        s = lax.axis_index("s")
        wid = c * NS + s

        def zero_step(i, _):
            dloc[pl.ds(i * LANES, LANES)] = jnp.zeros((LANES,), jnp.float32)
            return 0
        lax.fori_loop(0, N // LANES, zero_step, 0)

        lo, cnt = _split_blocks(nblk, NC * NS, wid)
        ones = jnp.ones((LANES,), jnp.float32)

        def step(i, _):
            pltpu.sync_copy(dst_hbm.at[pl.ds((lo + i) * EB, EB)], blk)
            for j in range(EB // LANES):
                idx = blk[pl.ds(j * LANES, LANES)]
                plsc.addupdate_scatter(dloc, [idx], ones)
            return 0
        lax.fori_loop(0, cnt, step, 0)

        pltpu.sync_copy(dloc, out_hbm.at[wid])

    return deg_kernel


def _make_agg(N, E, C, Fc):
    assert E % EB == 0 and N % NS == 0
    nblk = E // EB
    rpt = N // NS
    cpc = max(C // NC, 1)
    G = 4
    SG = 20
    assert nblk % SG == 0 and SG % G == 0
    NG = SG // G
    nsg = nblk // SG
    nworkers = NS if C >= 2 else NC * NS
    out_shape = (jax.ShapeDtypeStruct((C * N, Fc), jnp.float32) if C >= 2
                 else jax.ShapeDtypeStruct((NC, N, Fc), jnp.float32))
    mesh = plsc.VectorSubcoreMesh(core_axis_name="c", subcore_axis_name="s")

    @functools.partial(
        pl.kernel, mesh=mesh,
        out_type=out_shape,
        compiler_params=pltpu.CompilerParams(use_tc_tiling_on_sc=False),
        scratch_types=[
            pltpu.VMEM_SHARED((N, Fc), jnp.float32),
            pltpu.VMEM((SG, EB), jnp.int32),
            pltpu.VMEM((SG, EB), jnp.int32),
            pltpu.VMEM((2, G, EB, Fc), jnp.float32),
            pltpu.SemaphoreType.DMA,
            pltpu.SemaphoreType.DMA,
        ],
    )
    def agg_kernel(z_hbm, src_hbm, dst_hbm, y_hbm, acc, bidx, bdst,
                   rows, sem_g, sem_s):
        cid = lax.axis_index("c")
        sid = lax.axis_index("s")
        wid = sid if C >= 2 else cid * NS + sid
        slo, scnt = _split_blocks(nsg, nworkers, wid)

        for ci in range(cpc):
            if C >= 2:
                chunk = cid * cpc + ci
                row0 = chunk * N
                pltpu.sync_copy(z_hbm.at[pl.ds(row0 + sid * rpt, rpt)],
                                acc.at[pl.ds(sid * rpt, rpt)])
            else:
                row0 = jnp.int32(0)
                def zstep(i, _):
                    for j in range(Fc // LANES):
                        rows[0, 0, i, pl.ds(j * LANES, LANES)] = (
                            jnp.zeros((LANES,), jnp.float32))
                    return 0
                lax.fori_loop(0, EB, zstep, 0)
                nfull, tail = rpt // EB, rpt % EB
                for k in range(nfull):
                    pltpu.sync_copy(
                        rows.at[0, 0], acc.at[pl.ds(sid * rpt + k * EB, EB)])
                if tail:
                    pltpu.sync_copy(
                        rows.at[0, 0, pl.ds(0, tail)],
                        acc.at[pl.ds(sid * rpt + nfull * EB, tail)])

            plsc.subcore_barrier()

            def sg_step(t, _):
                b0 = (slo + t) * SG
                pltpu.sync_copy(src_hbm.at[pl.ds(b0, SG)], bidx)
                pltpu.sync_copy(dst_hbm.at[pl.ds(b0, SG)], bdst)
                if C >= 2:
                    for r in range(SG):
                        for k in range(EB // LANES):
                            sl = pl.ds(k * LANES, LANES)
                            bidx[r, sl] = bidx[r, sl] + row0
                pend = [[], []]
                for gg in range(NG):
                    bank = gg % 2
                    for d in pend[bank]:
                        d.wait()
                    pend[bank] = []
                    gd = [pltpu.async_copy(
                        z_hbm.at[bidx.at[gg * G + j]],
                        rows.at[bank, j], sem_g) for j in range(G)]
                    for j in range(G):
                        gd[j].wait()
                        pend[bank].append(pltpu.async_copy(
                            rows.at[bank, j], acc.at[bdst.at[gg * G + j]],
                            sem_s, add=True))
                for bank in range(2):
                    for d in pend[bank]:
                        d.wait()
                return 0
            lax.fori_loop(0, scnt, sg_step, 0)

            plsc.subcore_barrier()

            if C >= 2:
                pltpu.sync_copy(acc.at[pl.ds(sid * rpt, rpt)],
                                y_hbm.at[pl.ds(row0 + sid * rpt, rpt)])
            else:
                pltpu.sync_copy(acc.at[pl.ds(sid * rpt, rpt)],
                                y_hbm.at[cid, pl.ds(sid * rpt, rpt)])

    return agg_kernel


def _tc_call(body, grid, in_specs, out_specs, out_shape):
    return pl.pallas_call(
        body, grid=grid, in_specs=in_specs, out_specs=out_specs,
        out_shape=out_shape)


def _dinv_from(dp):
    return lax.rsqrt(jnp.sum(dp, axis=1, keepdims=True) + 1.0)


def _tc1(dpT, x, W1, N, R):
    H = W1.shape[1]
    Fc = H // 2

    def body(dp_ref, x_ref, w_ref, g1_ref):
        dv = _dinv_from(dp_ref[...])
        g = jnp.dot(x_ref[...] * dv, w_ref[...],
                    preferred_element_type=jnp.float32)
        g1_ref[0] = g[:, :Fc]
        g1_ref[1] = g[:, Fc:]

    return _tc_call(
        body, (N // R,),
        [pl.BlockSpec((R, NC * NS), lambda i: (i, 0)),
         pl.BlockSpec((R, x.shape[1]), lambda i: (i, 0)),
         pl.BlockSpec(W1.shape, lambda i: (0, 0))],
        pl.BlockSpec((2, R, Fc), lambda i: (0, i, 0)),
        jax.ShapeDtypeStruct((2, N, Fc), jnp.float32),
    )(dpT, x, W1)


def _tc2(a1, dpT, b1, N, R):
    Fc = a1.shape[2]
    b1c = b1.reshape(2, 1, Fc)

    def body(a_ref, dp_ref, b_ref, o_ref):
        dv = _dinv_from(dp_ref[...])
        for c in range(2):
            o_ref[c] = dv * jnp.maximum(dv * a_ref[c] + b_ref[c], 0.0)

    return _tc_call(
        body, (N // R,),
        [pl.BlockSpec((2, R, Fc), lambda i: (0, i, 0)),
         pl.BlockSpec((R, NC * NS), lambda i: (i, 0)),
         pl.BlockSpec((2, 1, Fc), lambda i: (0, 0, 0))],
        pl.BlockSpec((2, R, Fc), lambda i: (0, i, 0)),
        jax.ShapeDtypeStruct((2, N, Fc), jnp.float32),
    )(a1, dpT, b1c)


def _tc3(a2, dpT, W2, b2, W3, N, R):
    Fc_in = a2.shape[2]
    H3 = W3.shape[1]
    Fc = H3 // 8

    def body(a_ref, dp_ref, w2_ref, b2_ref, w3_ref, o_ref):
        dv = _dinv_from(dp_ref[...])
        h = jnp.concatenate([a_ref[0], a_ref[1]], axis=1) * dv
        t = jnp.maximum(jnp.dot(h, w2_ref[...],
                                preferred_element_type=jnp.float32)
                        + b2_ref[...], 0.0)
        g = jnp.dot(t * dv, w3_ref[...], preferred_element_type=jnp.float32)
        for c in range(8):
            o_ref[c] = g[:, c * Fc:(c + 1) * Fc]

    return _tc_call(
        body, (N // R,),
        [pl.BlockSpec((2, R, Fc_in), lambda i: (0, i, 0)),
         pl.BlockSpec((R, NC * NS), lambda i: (i, 0)),
         pl.BlockSpec(W2.shape, lambda i: (0, 0)),
         pl.BlockSpec((1, W2.shape[1]), lambda i: (0, 0)),
         pl.BlockSpec(W3.shape, lambda i: (0, 0))],
        pl.BlockSpec((8, R, Fc), lambda i: (0, i, 0)),
        jax.ShapeDtypeStruct((8, N, Fc), jnp.float32),
    )(a2, dpT, W2, b2.reshape(1, -1), W3)


def _tc4(a3, dpT, b3, W4, N, R):
    Fc = a3.shape[2]
    OUT = W4.shape[1]
    b3c = b3.reshape(8, 1, Fc)

    def body(a_ref, dp_ref, b_ref, w4_ref, o_ref):
        dv = _dinv_from(dp_ref[...])
        h = jnp.concatenate(
            [jnp.maximum(dv * a_ref[c] + b_ref[c], 0.0) for c in range(8)],
            axis=1)
        o_ref[...] = jnp.dot(h * dv, w4_ref[...],
                             preferred_element_type=jnp.float32)

    return _tc_call(
        body, (N // R,),
        [pl.BlockSpec((8, R, Fc), lambda i: (0, i, 0)),
         pl.BlockSpec((R, NC * NS), lambda i: (i, 0)),
         pl.BlockSpec((8, 1, Fc), lambda i: (0, 0, 0)),
         pl.BlockSpec(W4.shape, lambda i: (0, 0))],
        pl.BlockSpec((R, OUT), lambda i: (i, 0)),
        jax.ShapeDtypeStruct((N, OUT), jnp.float32),
    )(a3, dpT, b3c, W4)


def _tc5(parts, g4, dpT, b4, N, R):
    OUT = g4.shape[1]

    def body(p_ref, g_ref, dp_ref, b_ref, o_ref):
        dv = _dinv_from(dp_ref[...])
        o = dv * (p_ref[0] + p_ref[1] + g_ref[...]) + b_ref[...]
        m = jnp.max(o, axis=1, keepdims=True)
        e = o - m
        o_ref[...] = e - jnp.log(jnp.sum(jnp.exp(e), axis=1, keepdims=True))

    return _tc_call(
        body, (N // R,),
        [pl.BlockSpec((2, R, OUT), lambda i: (0, i, 0)),
         pl.BlockSpec((R, OUT), lambda i: (i, 0)),
         pl.BlockSpec((R, NC * NS), lambda i: (i, 0)),
         pl.BlockSpec((1, OUT), lambda i: (0, 0))],
        pl.BlockSpec((R, OUT), lambda i: (i, 0)),
        jax.ShapeDtypeStruct((N, OUT), jnp.float32),
    )(parts, g4, dpT, b4.reshape(1, -1))


def kernel(x, edge_index, W1, b1, W2, b2, W3, b3, W4, b4):
    N, DIN = x.shape
    E = edge_index.shape[1]
    R = ROWBLK
    src = edge_index[0]
    dst = edge_index[1]
    nblk = E // EB
    src2 = src.reshape(nblk, EB)
    dst2 = dst.reshape(nblk, EB)

    deg_parts = _make_degree(N, E)(dst)
    dpT = jnp.transpose(deg_parts)
    g1 = _tc1(dpT, x, W1, N, R)

    agg128 = _make_agg(N, E, 2, W1.shape[1] // 2)
    a1 = agg128(g1.reshape(2 * N, -1), src2, dst2)
    a1 = a1.reshape(2, N, -1)

    z2 = _tc2(a1, dpT, b1, N, R)
    a2 = agg128(z2.reshape(2 * N, -1), src2, dst2).reshape(2, N, -1)

    g3 = _tc3(a2, dpT, W2, b2, W3, N, R)
    agg512 = _make_agg(N, E, 8, W3.shape[1] // 8)
    a3 = agg512(g3.reshape(8 * N, -1), src2, dst2).reshape(8, N, -1)

    g4 = _tc4(a3, dpT, b3, W4, N, R)
    agg16 = _make_agg(N, E, 1, W4.shape[1])
    parts = agg16(g4, src2, dst2)

    return _tc5(parts, g4, dpT, b4, N, R)


# speedup vs baseline: 21.7590x; 1.0414x over previous
import functools

import jax
import jax.numpy as jnp
from jax import lax
from jax.experimental import pallas as pl
from jax.experimental.pallas import tpu as pltpu
from jax.experimental.pallas import tpu_sc as plsc

NC = 2
NS = 16
LANES = 16
EB = 128
ROWBLK = 1000


def _split_blocks(nblk, nworkers, w):
    base, extra = nblk // nworkers, nblk % nworkers
    cnt = base + jnp.where(w < extra, 1, 0)
    lo = w * base + jnp.minimum(w, extra)
    return lo, cnt


def _make_degree(N, E):
    assert E % EB == 0
    nblk = E // EB
    mesh = plsc.VectorSubcoreMesh(core_axis_name="c", subcore_axis_name="s")

    @functools.partial(
        pl.kernel, mesh=mesh,
        out_type=jax.ShapeDtypeStruct((NC * NS, N), jnp.float32),
        compiler_params=pltpu.CompilerParams(needs_layout_passes=False),
        scratch_types=[
            pltpu.VMEM((N,), jnp.float32),
            pltpu.VMEM((EB,), jnp.int32),
        ],
    )
    def deg_kernel(dst_hbm, out_hbm, dloc, blk):
        c = lax.axis_index("c")
        s = lax.axis_index("s")
        wid = c * NS + s

        def zero_step(i, _):
            dloc[pl.ds(i * LANES, LANES)] = jnp.zeros((LANES,), jnp.float32)
            return 0
        lax.fori_loop(0, N // LANES, zero_step, 0)

        lo, cnt = _split_blocks(nblk, NC * NS, wid)
        ones = jnp.ones((LANES,), jnp.float32)

        def step(i, _):
            pltpu.sync_copy(dst_hbm.at[pl.ds((lo + i) * EB, EB)], blk)
            for j in range(EB // LANES):
                idx = blk[pl.ds(j * LANES, LANES)]
                plsc.addupdate_scatter(dloc, [idx], ones)
            return 0
        lax.fori_loop(0, cnt, step, 0)

        pltpu.sync_copy(dloc, out_hbm.at[wid])

    return deg_kernel


def _make_agg(N, E, C, Fc):
    assert E % EB == 0 and N % NS == 0
    nblk = E // EB
    rpt = N // NS
    cpc = max(C // NC, 1)
    G = 5
    SG = 10
    assert nblk % SG == 0 and SG % G == 0
    NG = SG // G
    nsg = nblk // SG
    nworkers = NS if C >= 2 else NC * NS
    out_shape = (jax.ShapeDtypeStruct((C * N, Fc), jnp.float32) if C >= 2
                 else jax.ShapeDtypeStruct((NC, N, Fc), jnp.float32))
    mesh = plsc.VectorSubcoreMesh(core_axis_name="c", subcore_axis_name="s")

    @functools.partial(
        pl.kernel, mesh=mesh,
        out_type=out_shape,
        compiler_params=pltpu.CompilerParams(use_tc_tiling_on_sc=False),
        scratch_types=[
            pltpu.VMEM_SHARED((N, Fc), jnp.float32),
            pltpu.VMEM((2, SG, EB), jnp.int32),
            pltpu.VMEM((2, SG, EB), jnp.int32),
            pltpu.VMEM((2, G, EB, Fc), jnp.float32),
            pltpu.SemaphoreType.DMA,
            pltpu.SemaphoreType.DMA,
            pltpu.SemaphoreType.DMA,
            pltpu.SemaphoreType.DMA,
        ],
    )
    def agg_kernel(z_hbm, src_hbm, dst_hbm, y_hbm, acc, bidx, bdst,
                   rows, sem_g, sem_s, sem_i0, sem_i1):
        cid = lax.axis_index("c")
        sid = lax.axis_index("s")
        wid = sid if C >= 2 else cid * NS + sid
        slo, scnt = _split_blocks(nsg, nworkers, wid)
        isems = (sem_i0, sem_i1)

        def fire_idx(sgi, bank):
            b0 = sgi * SG
            pltpu.async_copy(src_hbm.at[pl.ds(b0, SG)], bidx.at[bank],
                             isems[bank])
            pltpu.async_copy(dst_hbm.at[pl.ds(b0, SG)], bdst.at[bank],
                             isems[bank])

        def wait_idx(bank):
            for _ in range(2):
                pltpu.make_async_copy(src_hbm.at[pl.ds(0, SG)],
                                      bidx.at[bank], isems[bank]).wait()

        for ci in range(cpc):
            if C >= 2:
                chunk = cid * cpc + ci
                row0 = chunk * N
                pltpu.sync_copy(z_hbm.at[pl.ds(row0 + sid * rpt, rpt)],
                                acc.at[pl.ds(sid * rpt, rpt)])
            else:
                row0 = jnp.int32(0)
                def zstep(i, _):
                    for j in range(Fc // LANES):
                        rows[0, 0, i, pl.ds(j * LANES, LANES)] = (
                            jnp.zeros((LANES,), jnp.float32))
                    return 0
                lax.fori_loop(0, EB, zstep, 0)
                nfull, tail = rpt // EB, rpt % EB
                for k in range(nfull):
                    pltpu.sync_copy(
                        rows.at[0, 0], acc.at[pl.ds(sid * rpt + k * EB, EB)])
                if tail:
                    pltpu.sync_copy(
                        rows.at[0, 0, pl.ds(0, tail)],
                        acc.at[pl.ds(sid * rpt + nfull * EB, tail)])

            plsc.subcore_barrier()

            def process(bank):
                if C >= 2:
                    for r in range(SG):
                        for k in range(EB // LANES):
                            sl = pl.ds(k * LANES, LANES)
                            bidx[bank, r, sl] = bidx[bank, r, sl] + row0
                pend = [[], []]
                for gg in range(NG):
                    rb = gg % 2
                    for d in pend[rb]:
                        d.wait()
                    pend[rb] = []
                    gd = [pltpu.async_copy(
                        z_hbm.at[bidx.at[bank, gg * G + j]],
                        rows.at[rb, j], sem_g) for j in range(G)]
                    for j in range(G):
                        gd[j].wait()
                        pend[rb].append(pltpu.async_copy(
                            rows.at[rb, j],
                            acc.at[bdst.at[bank, gg * G + j]],
                            sem_s, add=True))
                for rb in range(2):
                    for d in pend[rb]:
                        d.wait()

            @pl.when(scnt > 0)
            def _():
                fire_idx(slo, 0)

            def pair(p, _):
                sg0 = slo + 2 * p

                @pl.when(2 * p + 1 < scnt)
                def _():
                    fire_idx(sg0 + 1, 1)
                wait_idx(0)
                process(0)

                @pl.when(2 * p + 1 < scnt)
                def _():
                    @pl.when(2 * p + 2 < scnt)
                    def _():
                        fire_idx(sg0 + 2, 0)
                    wait_idx(1)
                    process(1)
                return 0
            lax.fori_loop(0, (scnt + 1) // 2, pair, 0)

            plsc.subcore_barrier()

            if C >= 2:
                pltpu.sync_copy(acc.at[pl.ds(sid * rpt, rpt)],
                                y_hbm.at[pl.ds(row0 + sid * rpt, rpt)])
            else:
                pltpu.sync_copy(acc.at[pl.ds(sid * rpt, rpt)],
                                y_hbm.at[cid, pl.ds(sid * rpt, rpt)])

    return agg_kernel


def _tc_call(body, grid, in_specs, out_specs, out_shape):
    return pl.pallas_call(
        body, grid=grid, in_specs=in_specs, out_specs=out_specs,
        out_shape=out_shape)


def _dinv_from(dp):
    return lax.rsqrt(jnp.sum(dp, axis=1, keepdims=True) + 1.0)


def _tc1(dpT, x, W1, N, R):
    H = W1.shape[1]
    Fc = H // 2

    def body(dp_ref, x_ref, w_ref, g1_ref):
        dv = _dinv_from(dp_ref[...])
        g = jnp.dot(x_ref[...] * dv, w_ref[...],
                    preferred_element_type=jnp.float32)
        g1_ref[0] = g[:, :Fc]
        g1_ref[1] = g[:, Fc:]

    return _tc_call(
        body, (N // R,),
        [pl.BlockSpec((R, NC * NS), lambda i: (i, 0)),
         pl.BlockSpec((R, x.shape[1]), lambda i: (i, 0)),
         pl.BlockSpec(W1.shape, lambda i: (0, 0))],
        pl.BlockSpec((2, R, Fc), lambda i: (0, i, 0)),
        jax.ShapeDtypeStruct((2, N, Fc), jnp.float32),
    )(dpT, x, W1)


def _tc2(a1, dpT, b1, N, R):
    Fc = a1.shape[2]
    b1c = b1.reshape(2, 1, Fc)

    def body(a_ref, dp_ref, b_ref, o_ref):
        dv = _dinv_from(dp_ref[...])
        for c in range(2):
            o_ref[c] = dv * jnp.maximum(dv * a_ref[c] + b_ref[c], 0.0)

    return _tc_call(
        body, (N // R,),
        [pl.BlockSpec((2, R, Fc), lambda i: (0, i, 0)),
         pl.BlockSpec((R, NC * NS), lambda i: (i, 0)),
         pl.BlockSpec((2, 1, Fc), lambda i: (0, 0, 0))],
        pl.BlockSpec((2, R, Fc), lambda i: (0, i, 0)),
        jax.ShapeDtypeStruct((2, N, Fc), jnp.float32),
    )(a1, dpT, b1c)


def _tc3(a2, dpT, W2, b2, W3, N, R):
    Fc_in = a2.shape[2]
    H3 = W3.shape[1]
    Fc = H3 // 8

    def body(a_ref, dp_ref, w2_ref, b2_ref, w3_ref, o_ref):
        dv = _dinv_from(dp_ref[...])
        h = jnp.concatenate([a_ref[0], a_ref[1]], axis=1) * dv
        t = jnp.maximum(jnp.dot(h, w2_ref[...],
                                preferred_element_type=jnp.float32)
                        + b2_ref[...], 0.0)
        g = jnp.dot(t * dv, w3_ref[...], preferred_element_type=jnp.float32)
        for c in range(8):
            o_ref[c] = g[:, c * Fc:(c + 1) * Fc]

    return _tc_call(
        body, (N // R,),
        [pl.BlockSpec((2, R, Fc_in), lambda i: (0, i, 0)),
         pl.BlockSpec((R, NC * NS), lambda i: (i, 0)),
         pl.BlockSpec(W2.shape, lambda i: (0, 0)),
         pl.BlockSpec((1, W2.shape[1]), lambda i: (0, 0)),
         pl.BlockSpec(W3.shape, lambda i: (0, 0))],
        pl.BlockSpec((8, R, Fc), lambda i: (0, i, 0)),
        jax.ShapeDtypeStruct((8, N, Fc), jnp.float32),
    )(a2, dpT, W2, b2.reshape(1, -1), W3)


def _tc4(a3, dpT, b3, W4, N, R):
    Fc = a3.shape[2]
    OUT = W4.shape[1]
    b3c = b3.reshape(8, 1, Fc)

    def body(a_ref, dp_ref, b_ref, w4_ref, o_ref):
        dv = _dinv_from(dp_ref[...])
        h = jnp.concatenate(
            [jnp.maximum(dv * a_ref[c] + b_ref[c], 0.0) for c in range(8)],
            axis=1)
        o_ref[...] = jnp.dot(h * dv, w4_ref[...],
                             preferred_element_type=jnp.float32)

    return _tc_call(
        body, (N // R,),
        [pl.BlockSpec((8, R, Fc), lambda i: (0, i, 0)),
         pl.BlockSpec((R, NC * NS), lambda i: (i, 0)),
         pl.BlockSpec((8, 1, Fc), lambda i: (0, 0, 0)),
         pl.BlockSpec(W4.shape, lambda i: (0, 0))],
        pl.BlockSpec((R, OUT), lambda i: (i, 0)),
        jax.ShapeDtypeStruct((N, OUT), jnp.float32),
    )(a3, dpT, b3c, W4)


def _tc5(parts, g4, dpT, b4, N, R):
    OUT = g4.shape[1]

    def body(p_ref, g_ref, dp_ref, b_ref, o_ref):
        dv = _dinv_from(dp_ref[...])
        o = dv * (p_ref[0] + p_ref[1] + g_ref[...]) + b_ref[...]
        m = jnp.max(o, axis=1, keepdims=True)
        e = o - m
        o_ref[...] = e - jnp.log(jnp.sum(jnp.exp(e), axis=1, keepdims=True))

    return _tc_call(
        body, (N // R,),
        [pl.BlockSpec((2, R, OUT), lambda i: (0, i, 0)),
         pl.BlockSpec((R, OUT), lambda i: (i, 0)),
         pl.BlockSpec((R, NC * NS), lambda i: (i, 0)),
         pl.BlockSpec((1, OUT), lambda i: (0, 0))],
        pl.BlockSpec((R, OUT), lambda i: (i, 0)),
        jax.ShapeDtypeStruct((N, OUT), jnp.float32),
    )(parts, g4, dpT, b4.reshape(1, -1))


def kernel(x, edge_index, W1, b1, W2, b2, W3, b3, W4, b4):
    N, DIN = x.shape
    E = edge_index.shape[1]
    R = ROWBLK
    src = edge_index[0]
    dst = edge_index[1]
    nblk = E // EB
    src2 = src.reshape(nblk, EB)
    dst2 = dst.reshape(nblk, EB)

    deg_parts = _make_degree(N, E)(dst)
    dpT = jnp.transpose(deg_parts)
    g1 = _tc1(dpT, x, W1, N, R)

    agg128 = _make_agg(N, E, 2, W1.shape[1] // 2)
    a1 = agg128(g1.reshape(2 * N, -1), src2, dst2)
    a1 = a1.reshape(2, N, -1)

    z2 = _tc2(a1, dpT, b1, N, R)
    a2 = agg128(z2.reshape(2 * N, -1), src2, dst2).reshape(2, N, -1)

    g3 = _tc3(a2, dpT, W2, b2, W3, N, R)
    agg512 = _make_agg(N, E, 8, W3.shape[1] // 8)
    a3 = agg512(g3.reshape(8 * N, -1), src2, dst2).reshape(8, N, -1)

    g4 = _tc4(a3, dpT, b3, W4, N, R)
    agg16 = _make_agg(N, E, 1, W4.shape[1])
    parts = agg16(g4, src2, dst2)

    return _tc5(parts, g4, dpT, b4, N, R)
